# baseline (speedup 1.0000x reference)
import functools
import jax
import jax.numpy as jnp
from jax import lax
from jax.experimental import pallas as pl
from jax.experimental.pallas import tpu as pltpu
from jax.experimental.pallas import tpu_sc as plsc

B, N, M, C, K = 2, 4096, 4096, 64, 32
R2 = 0.04
RADIUS = 0.2
TN = 128
TE = TN * K
D = 128
NC, NS = 2, 16
NW = NC * NS


def _p1_body(q_ref, sT_ref, m_ref, idx_ref, nm_ref, nn_ref):
    b = pl.program_id(0)
    q = q_ref[0]
    sT = sT_ref[0]
    msk = m_ref[0]

    acc = jnp.zeros((TN, M), jnp.float32)
    for d in range(3):
        diff = q[:, d:d + 1] - sT[d:d + 1, :]
        acc = acc + diff * diff

    maskb = msk > 0.0
    valid = (acc < R2) & maskb

    iota = lax.broadcasted_iota(jnp.int32, (TN, M), 1)
    dm = jnp.where(maskb, acc, jnp.inf)
    minv = jnp.min(dm, axis=1, keepdims=True)
    nn = jnp.min(jnp.where(dm == minv, iota, M), axis=1, keepdims=True)
    nn_ref[0] = nn + b * M

    iotaK = lax.broadcasted_iota(jnp.int32, (1, K), 1)
    vf = valid.astype(jnp.float32)
    v2 = vf.reshape(TN * 32, 128)
    ioA = lax.broadcasted_iota(jnp.int32, (128, 128), 0)
    ioB = lax.broadcasted_iota(jnp.int32, (128, 128), 1)
    U128 = (ioA <= ioB).astype(jnp.float32)
    wr = jnp.dot(v2, U128, preferred_element_type=jnp.float32)
    t = wr[:, 127:128].reshape(TN, 32)
    ioC = lax.broadcasted_iota(jnp.int32, (32, 32), 0)
    ioD = lax.broadcasted_iota(jnp.int32, (32, 32), 1)
    SU = (ioC < ioD).astype(jnp.float32)
    offs = jnp.dot(t, SU, preferred_element_type=jnp.float32)
    rank3 = wr.reshape(TN, 32, 128) + offs[:, :, None]
    crb = jnp.minimum(rank3, 34.0).astype(jnp.bfloat16)
    one_b = jnp.bfloat16(1.0)
    zero_b = jnp.bfloat16(0.0)
    idxs = jnp.zeros((TN, K), jnp.int32)
    for k in range(K):
        part = jnp.sum(jnp.where(crb <= jnp.bfloat16(k), one_b, zero_b),
                       axis=2)
        cnt = jnp.sum(part.astype(jnp.float32), axis=1,
                      keepdims=True).astype(jnp.int32)
        idxs = idxs + jnp.where(iotaK == k, cnt, 0)

    nmv = idxs < M
    safe0 = jnp.where(nmv[:, 0:1], idxs[:, 0:1], 0)
    idxf = jnp.where(nmv, idxs, safe0)
    idx_ref[0] = idxf + b * M
    nm_ref[0] = nmv.astype(jnp.float32)


def _run_p1(query_xyz, sxyzT, maskR):
    grid = (B, N // TN)
    return pl.pallas_call(
        _p1_body,
        grid=grid,
        in_specs=[
            pl.BlockSpec((1, TN, 3), lambda b, i: (b, i, 0)),
            pl.BlockSpec((1, 3, M), lambda b, i: (b, 0, 0)),
            pl.BlockSpec((1, 1, M), lambda b, i: (b, 0, 0)),
        ],
        out_specs=[
            pl.BlockSpec((1, TN, K), lambda b, i: (b, i, 0)),
            pl.BlockSpec((1, TN, K), lambda b, i: (b, i, 0)),
            pl.BlockSpec((1, TN, 1), lambda b, i: (b, i, 0)),
        ],
        out_shape=[
            jax.ShapeDtypeStruct((B, N, K), jnp.int32),
            jax.ShapeDtypeStruct((B, N, K), jnp.float32),
            jax.ShapeDtypeStruct((B, N, 1), jnp.int32),
        ],
    )(query_xyz, sxyzT, maskR)


def _make_sc_gather(n_rows, chunk):
    n_iters = n_rows // (NW * chunk)
    per_w = n_rows // NW
    mesh = plsc.VectorSubcoreMesh(core_axis_name="c", subcore_axis_name="s")

    @functools.partial(
        pl.kernel, mesh=mesh,
        out_type=jax.ShapeDtypeStruct((n_rows, D), jnp.float32),
        scratch_types=[
            pltpu.VMEM((per_w,), jnp.int32),
            pltpu.VMEM((chunk, D), jnp.float32),
            pltpu.SemaphoreType.DMA,
        ],
    )
    def gather(table_hbm, idx_hbm, out_hbm, idx_v, rows_v, sem):
        wid = lax.axis_index("s") * NC + lax.axis_index("c")
        base = wid * per_w
        pltpu.sync_copy(idx_hbm.at[pl.ds(base, per_w)], idx_v)
        for j in range(n_iters):
            off = j * chunk
            pltpu.async_copy(
                table_hbm.at[idx_v.at[pl.ds(off, chunk)]], rows_v, sem
            ).wait()
            pltpu.sync_copy(rows_v, out_hbm.at[pl.ds(base + off, chunk)])

    return gather


def _p3a_body(ge_ref, q_ref, wT_ref, be_ref, o_ref):
    rows = ge_ref[0]
    q = q_ref[0]
    xyz = rows[:, :, 64:67]
    pos = (xyz - q[:, None, :]) / RADIUS
    pos2 = pos.reshape(TE, 3)
    y1 = jnp.dot(pos2, wT_ref[...], preferred_element_type=jnp.float32) + be_ref[...]
    s1 = jnp.sum(y1, axis=0, keepdims=True)
    s2 = jnp.sum(y1 * y1, axis=0, keepdims=True)
    o_ref[0, 0] = jnp.concatenate([s1, s2], axis=0)


def _run_p3a(ge, query_xyz, WeffT, beff):
    grid = (B, N // TN)
    return pl.pallas_call(
        _p3a_body,
        grid=grid,
        in_specs=[
            pl.BlockSpec((1, TN, K, D), lambda b, i: (b, i, 0, 0)),
            pl.BlockSpec((1, TN, 3), lambda b, i: (b, i, 0)),
            pl.BlockSpec((3, C), lambda b, i: (0, 0)),
            pl.BlockSpec((1, C), lambda b, i: (0, 0)),
        ],
        out_specs=pl.BlockSpec((1, 1, 2, C), lambda b, i: (b, i, 0, 0)),
        out_shape=jax.ShapeDtypeStruct((B, N // TN, 2, C), jnp.float32),
    )(ge.reshape(B, N, K, D), query_xyz, WeffT, beff)


def _p3b_body(ge_ref, gi_ref, q_ref, a1T_ref, c1_ref, phiT_ref, bphi_ref,
              psiT_ref, bpsi_ref, geffT_ref, geffb_ref, o_ref):
    rows = ge_ref[0]
    q = q_ref[0]
    xyz = rows[:, :, 64:67]
    pos = ((xyz - q[:, None, :]) / RADIUS).reshape(TE, 3)
    delta = jnp.maximum(
        jnp.dot(pos, a1T_ref[...], preferred_element_type=jnp.float32)
        + c1_ref[...], 0.0)
    xj = rows[:, :, 0:64].reshape(TE, C)
    lin_j = jnp.dot(xj, psiT_ref[...], preferred_element_type=jnp.float32) + bpsi_ref[...]
    xi = gi_ref[0][:, 0:64]
    lin_i = jnp.dot(xi, phiT_ref[...], preferred_element_type=jnp.float32) + bphi_ref[...]
    rel3 = lin_i[:, None, :] - lin_j.reshape(TN, K, C) + delta.reshape(TN, K, C)
    rel = rel3.reshape(TE, C)
    y2 = jnp.dot(rel, geffT_ref[...], preferred_element_type=jnp.float32) + geffb_ref[...]
    s1 = jnp.sum(y2, axis=0, keepdims=True)
    s2 = jnp.sum(y2 * y2, axis=0, keepdims=True)
    o_ref[0, 0] = jnp.concatenate([s1, s2], axis=0)


def _run_p3b(ge, gi, query_xyz, A1T, C1, phiT, bphi, psiT, bpsi, GeffT, geffb):
    grid = (B, N // TN)
    vec = lambda b, i: (0, 0)
    return pl.pallas_call(
        _p3b_body,
        grid=grid,
        in_specs=[
            pl.BlockSpec((1, TN, K, D), lambda b, i: (b, i, 0, 0)),
            pl.BlockSpec((1, TN, D), lambda b, i: (b, i, 0)),
            pl.BlockSpec((1, TN, 3), lambda b, i: (b, i, 0)),
            pl.BlockSpec((3, C), vec),
            pl.BlockSpec((1, C), vec),
            pl.BlockSpec((C, C), vec),
            pl.BlockSpec((1, C), vec),
            pl.BlockSpec((C, C), vec),
            pl.BlockSpec((1, C), vec),
            pl.BlockSpec((C, C), vec),
            pl.BlockSpec((1, C), vec),
        ],
        out_specs=pl.BlockSpec((1, 1, 2, C), lambda b, i: (b, i, 0, 0)),
        out_shape=jax.ShapeDtypeStruct((B, N // TN, 2, C), jnp.float32),
    )(ge.reshape(B, N, K, D), gi.reshape(B, N, D), query_xyz,
      A1T, C1, phiT, bphi, psiT, bpsi, GeffT, geffb)


def _p3c_body(ge_ref, gi_ref, q_ref, qm_ref, nm_ref, a1T_ref, c1_ref,
              phiT_ref, bphi_ref, psiT_ref, bpsi_ref, a2T_ref, c2_ref,
              alphaT_ref, balpha_ref, o_ref):
    rows = ge_ref[0]
    q = q_ref[0]
    xyz = rows[:, :, 64:67]
    pos = ((xyz - q[:, None, :]) / RADIUS).reshape(TE, 3)
    delta = jnp.maximum(
        jnp.dot(pos, a1T_ref[...], preferred_element_type=jnp.float32)
        + c1_ref[...], 0.0)
    xj = rows[:, :, 0:64].reshape(TE, C)
    lin_j = jnp.dot(xj, psiT_ref[...], preferred_element_type=jnp.float32) + bpsi_ref[...]
    xi = gi_ref[0][:, 0:64]
    lin_i = jnp.dot(xi, phiT_ref[...], preferred_element_type=jnp.float32) + bphi_ref[...]
    rel3 = lin_i[:, None, :] - lin_j.reshape(TN, K, C) + delta.reshape(TN, K, C)
    rel = rel3.reshape(TE, C)
    s = jnp.maximum(
        jnp.dot(rel, a2T_ref[...], preferred_element_type=jnp.float32)
        + c2_ref[...], 0.0)
    s3 = s.reshape(TN, K, C)
    mx = jnp.max(s3, axis=1, keepdims=True)
    e = jnp.exp(s3 - mx)
    w = e / jnp.sum(e, axis=1, keepdims=True)
    feats = (jnp.dot(xj, alphaT_ref[...], preferred_element_type=jnp.float32)
             + balpha_ref[...] + delta).reshape(TN, K, C)
    fm = nm_ref[0] + (1.0 - qm_ref[0])
    feats = feats * fm[:, :, None]
    o_ref[0] = jnp.sum(w * feats, axis=1)


def _run_p3c(ge, gi, query_xyz, qmaskR, nmask, A1T, C1, phiT, bphi, psiT,
             bpsi, A2T, C2, alphaT, balpha):
    grid = (B, N // TN)
    vec = lambda b, i: (0, 0)
    return pl.pallas_call(
        _p3c_body,
        grid=grid,
        in_specs=[
            pl.BlockSpec((1, TN, K, D), lambda b, i: (b, i, 0, 0)),
            pl.BlockSpec((1, TN, D), lambda b, i: (b, i, 0)),
            pl.BlockSpec((1, TN, 3), lambda b, i: (b, i, 0)),
            pl.BlockSpec((1, TN, 1), lambda b, i: (b, i, 0)),
            pl.BlockSpec((1, TN, K), lambda b, i: (b, i, 0)),
            pl.BlockSpec((3, C), vec),
            pl.BlockSpec((1, C), vec),
            pl.BlockSpec((C, C), vec),
            pl.BlockSpec((1, C), vec),
            pl.BlockSpec((C, C), vec),
            pl.BlockSpec((1, C), vec),
            pl.BlockSpec((C, C), vec),
            pl.BlockSpec((1, C), vec),
            pl.BlockSpec((C, C), vec),
            pl.BlockSpec((1, C), vec),
        ],
        out_specs=pl.BlockSpec((1, TN, C), lambda b, i: (b, i, 0)),
        out_shape=jax.ShapeDtypeStruct((B, N, C), jnp.float32),
    )(ge.reshape(B, N, K, D), gi.reshape(B, N, D), query_xyz, qmaskR, nmask,
      A1T, C1, phiT, bphi, psiT, bpsi, A2T, C2, alphaT, balpha)


def _bn_affine(sums, count, g, b):
    mu = sums[0] / count
    var = sums[1] / count - mu * mu
    a = g / jnp.sqrt(var + 1e-5)
    return a, mu


@jax.jit
def kernel(query_xyz, support_xyz, query_mask, support_mask, support_features,
           W_theta1, b_theta1, W_theta2, b_theta2, g_bn_theta, b_bn_theta,
           W_phi, b_phi, W_psi, b_psi, W_alpha, b_alpha,
           W_g1, b_g1, W_g2, b_g2, g_bn_g, b_bn_g):
    f32 = jnp.float32
    sxyzT = jnp.transpose(support_xyz, (0, 2, 1))
    maskR = support_mask.reshape(B, 1, M)
    Weff = jnp.dot(W_theta2, W_theta1)
    beff = jnp.dot(W_theta2, b_theta1) + b_theta2
    Geff = jnp.dot(W_g2, W_g1)
    geffb = jnp.dot(W_g2, b_g1) + b_g2

    idxg, nmask, nng = _run_p1(query_xyz, sxyzT, maskR)

    table = jnp.concatenate(
        [jnp.transpose(support_features, (0, 2, 1)),
         support_xyz,
         jnp.zeros((B, M, D - C - 3), f32)], axis=-1).reshape(B * M, D)
    ge = _make_sc_gather(B * N * K, 512)(table, idxg.reshape(-1))
    gi = _make_sc_gather(B * N, 256)(table, nng.reshape(-1))

    cnt_e = float(B * N * K)
    p1 = _run_p3a(ge, query_xyz, Weff.T, beff.reshape(1, C))
    s = p1.sum(axis=(0, 1))
    a1, mu1 = _bn_affine(s, cnt_e, g_bn_theta, None)
    A1 = a1[:, None] * Weff
    C1 = (a1 * (beff - mu1) + b_bn_theta).reshape(1, C)

    p2 = _run_p3b(ge, gi, query_xyz, A1.T, C1, W_phi.T, b_phi.reshape(1, C),
                  W_psi.T, b_psi.reshape(1, C), Geff.T, geffb.reshape(1, C))
    s2 = p2.sum(axis=(0, 1))
    a2, mu2 = _bn_affine(s2, cnt_e, g_bn_g, None)
    A2 = a2[:, None] * Geff
    C2 = (a2 * (geffb - mu2) + b_bn_g).reshape(1, C)

    out = _run_p3c(ge, gi, query_xyz, query_mask.reshape(B, N, 1), nmask,
                   A1.T, C1, W_phi.T, b_phi.reshape(1, C),
                   W_psi.T, b_psi.reshape(1, C), A2.T, C2,
                   W_alpha.T, b_alpha.reshape(1, C))
    return jnp.transpose(out, (0, 2, 1))


# speedup vs baseline: 1.6852x; 1.6852x over previous
import functools
import jax
import jax.numpy as jnp
from jax import lax
from jax.experimental import pallas as pl
from jax.experimental.pallas import tpu as pltpu
from jax.experimental.pallas import tpu_sc as plsc

B, N, M, C, K = 2, 4096, 4096, 64, 32
R2 = 0.04
RADIUS = 0.2
TN = 128
TE = TN * K
D = 128
NC, NS = 2, 16
NW = NC * NS


def _p1_body(q_ref, sT_ref, m_ref, idx_ref, nm_ref, nn_ref):
    b = pl.program_id(0)
    q = q_ref[0]
    sT = sT_ref[0]
    msk = m_ref[0]

    acc = jnp.zeros((TN, M), jnp.float32)
    for d in range(3):
        diff = q[:, d:d + 1] - sT[d:d + 1, :]
        acc = acc + diff * diff

    maskb = msk > 0.0
    valid = (acc < R2) & maskb

    iota = lax.broadcasted_iota(jnp.int32, (TN, M), 1)
    dm = jnp.where(maskb, acc, jnp.inf)
    minv = jnp.min(dm, axis=1, keepdims=True)
    nn = jnp.min(jnp.where(dm == minv, iota, M), axis=1, keepdims=True)
    nn_ref[0] = nn + b * M

    iotaK = lax.broadcasted_iota(jnp.int32, (1, K), 1)
    vf = valid.astype(jnp.float32)
    v2 = vf.reshape(TN * 32, 128)
    ioA = lax.broadcasted_iota(jnp.int32, (128, 128), 0)
    ioB = lax.broadcasted_iota(jnp.int32, (128, 128), 1)
    U128 = (ioA <= ioB).astype(jnp.float32)
    wr = jnp.dot(v2, U128, preferred_element_type=jnp.float32)
    t = wr[:, 127:128].reshape(TN, 32)
    ioC = lax.broadcasted_iota(jnp.int32, (32, 32), 0)
    ioD = lax.broadcasted_iota(jnp.int32, (32, 32), 1)
    SU = (ioC < ioD).astype(jnp.float32)
    offs = jnp.dot(t, SU, preferred_element_type=jnp.float32)
    rank = (wr.reshape(TN, 32, 128) + offs[:, :, None]).reshape(TN, M)
    idxs = jnp.zeros((TN, K), jnp.int32)
    for k in range(K):
        cnt = jnp.sum(jnp.where(rank <= float(k), 1.0, 0.0), axis=1,
                      keepdims=True).astype(jnp.int32)
        idxs = idxs + jnp.where(iotaK == k, cnt, 0)

    nmv = idxs < M
    safe0 = jnp.where(nmv[:, 0:1], idxs[:, 0:1], 0)
    idxf = jnp.where(nmv, idxs, safe0)
    idx_ref[0] = idxf + b * M
    nm_ref[0] = nmv.astype(jnp.float32)


def _run_p1(query_xyz, sxyzT, maskR):
    grid = (B, N // TN)
    return pl.pallas_call(
        _p1_body,
        grid=grid,
        in_specs=[
            pl.BlockSpec((1, TN, 3), lambda b, i: (b, i, 0)),
            pl.BlockSpec((1, 3, M), lambda b, i: (b, 0, 0)),
            pl.BlockSpec((1, 1, M), lambda b, i: (b, 0, 0)),
        ],
        out_specs=[
            pl.BlockSpec((1, TN, K), lambda b, i: (b, i, 0)),
            pl.BlockSpec((1, TN, K), lambda b, i: (b, i, 0)),
            pl.BlockSpec((1, TN, 1), lambda b, i: (b, i, 0)),
        ],
        out_shape=[
            jax.ShapeDtypeStruct((B, N, K), jnp.int32),
            jax.ShapeDtypeStruct((B, N, K), jnp.float32),
            jax.ShapeDtypeStruct((B, N, 1), jnp.int32),
        ],
    )(query_xyz, sxyzT, maskR)


def _make_sc_gather(n_rows, chunk):
    n_iters = n_rows // (NW * chunk)
    per_w = n_rows // NW
    mesh = plsc.VectorSubcoreMesh(core_axis_name="c", subcore_axis_name="s")

    @functools.partial(
        pl.kernel, mesh=mesh,
        out_type=jax.ShapeDtypeStruct((n_rows, D), jnp.float32),
        scratch_types=[
            pltpu.VMEM((per_w,), jnp.int32),
            pltpu.VMEM((chunk, D), jnp.float32),
            pltpu.SemaphoreType.DMA,
        ],
    )
    def gather(table_hbm, idx_hbm, out_hbm, idx_v, rows_v, sem):
        wid = lax.axis_index("s") * NC + lax.axis_index("c")
        base = wid * per_w
        pltpu.sync_copy(idx_hbm.at[pl.ds(base, per_w)], idx_v)
        for j in range(n_iters):
            off = j * chunk
            pltpu.async_copy(
                table_hbm.at[idx_v.at[pl.ds(off, chunk)]], rows_v, sem
            ).wait()
            pltpu.sync_copy(rows_v, out_hbm.at[pl.ds(base + off, chunk)])

    return gather


def _p3a_body(ge_ref, q_ref, wT_ref, be_ref, o_ref):
    rows = ge_ref[0]
    q = q_ref[0]
    xyz = rows[:, :, 64:67]
    pos = (xyz - q[:, None, :]) / RADIUS
    pos2 = pos.reshape(TE, 3)
    y1 = jnp.dot(pos2, wT_ref[...], preferred_element_type=jnp.float32) + be_ref[...]
    s1 = jnp.sum(y1, axis=0, keepdims=True)
    s2 = jnp.sum(y1 * y1, axis=0, keepdims=True)
    o_ref[0, 0] = jnp.concatenate([s1, s2], axis=0)


def _run_p3a(ge, query_xyz, WeffT, beff):
    grid = (B, N // TN)
    return pl.pallas_call(
        _p3a_body,
        grid=grid,
        in_specs=[
            pl.BlockSpec((1, TN, K, D), lambda b, i: (b, i, 0, 0)),
            pl.BlockSpec((1, TN, 3), lambda b, i: (b, i, 0)),
            pl.BlockSpec((3, C), lambda b, i: (0, 0)),
            pl.BlockSpec((1, C), lambda b, i: (0, 0)),
        ],
        out_specs=pl.BlockSpec((1, 1, 2, C), lambda b, i: (b, i, 0, 0)),
        out_shape=jax.ShapeDtypeStruct((B, N // TN, 2, C), jnp.float32),
    )(ge.reshape(B, N, K, D), query_xyz, WeffT, beff)


def _p3b_body(ge_ref, gi_ref, q_ref, a1T_ref, c1_ref, phiT_ref, bphi_ref,
              psiT_ref, bpsi_ref, geffT_ref, geffb_ref, o_ref):
    rows = ge_ref[0]
    q = q_ref[0]
    xyz = rows[:, :, 64:67]
    pos = ((xyz - q[:, None, :]) / RADIUS).reshape(TE, 3)
    delta = jnp.maximum(
        jnp.dot(pos, a1T_ref[...], preferred_element_type=jnp.float32)
        + c1_ref[...], 0.0)
    xj = rows[:, :, 0:64].reshape(TE, C)
    lin_j = jnp.dot(xj, psiT_ref[...], preferred_element_type=jnp.float32) + bpsi_ref[...]
    xi = gi_ref[0][:, 0:64]
    lin_i = jnp.dot(xi, phiT_ref[...], preferred_element_type=jnp.float32) + bphi_ref[...]
    rel3 = lin_i[:, None, :] - lin_j.reshape(TN, K, C) + delta.reshape(TN, K, C)
    rel = rel3.reshape(TE, C)
    y2 = jnp.dot(rel, geffT_ref[...], preferred_element_type=jnp.float32) + geffb_ref[...]
    s1 = jnp.sum(y2, axis=0, keepdims=True)
    s2 = jnp.sum(y2 * y2, axis=0, keepdims=True)
    o_ref[0, 0] = jnp.concatenate([s1, s2], axis=0)


def _run_p3b(ge, gi, query_xyz, A1T, C1, phiT, bphi, psiT, bpsi, GeffT, geffb):
    grid = (B, N // TN)
    vec = lambda b, i: (0, 0)
    return pl.pallas_call(
        _p3b_body,
        grid=grid,
        in_specs=[
            pl.BlockSpec((1, TN, K, D), lambda b, i: (b, i, 0, 0)),
            pl.BlockSpec((1, TN, D), lambda b, i: (b, i, 0)),
            pl.BlockSpec((1, TN, 3), lambda b, i: (b, i, 0)),
            pl.BlockSpec((3, C), vec),
            pl.BlockSpec((1, C), vec),
            pl.BlockSpec((C, C), vec),
            pl.BlockSpec((1, C), vec),
            pl.BlockSpec((C, C), vec),
            pl.BlockSpec((1, C), vec),
            pl.BlockSpec((C, C), vec),
            pl.BlockSpec((1, C), vec),
        ],
        out_specs=pl.BlockSpec((1, 1, 2, C), lambda b, i: (b, i, 0, 0)),
        out_shape=jax.ShapeDtypeStruct((B, N // TN, 2, C), jnp.float32),
    )(ge.reshape(B, N, K, D), gi.reshape(B, N, D), query_xyz,
      A1T, C1, phiT, bphi, psiT, bpsi, GeffT, geffb)


def _p3c_body(ge_ref, gi_ref, q_ref, qm_ref, nm_ref, a1T_ref, c1_ref,
              phiT_ref, bphi_ref, psiT_ref, bpsi_ref, a2T_ref, c2_ref,
              alphaT_ref, balpha_ref, o_ref):
    rows = ge_ref[0]
    q = q_ref[0]
    xyz = rows[:, :, 64:67]
    pos = ((xyz - q[:, None, :]) / RADIUS).reshape(TE, 3)
    delta = jnp.maximum(
        jnp.dot(pos, a1T_ref[...], preferred_element_type=jnp.float32)
        + c1_ref[...], 0.0)
    xj = rows[:, :, 0:64].reshape(TE, C)
    lin_j = jnp.dot(xj, psiT_ref[...], preferred_element_type=jnp.float32) + bpsi_ref[...]
    xi = gi_ref[0][:, 0:64]
    lin_i = jnp.dot(xi, phiT_ref[...], preferred_element_type=jnp.float32) + bphi_ref[...]
    rel3 = lin_i[:, None, :] - lin_j.reshape(TN, K, C) + delta.reshape(TN, K, C)
    rel = rel3.reshape(TE, C)
    s = jnp.maximum(
        jnp.dot(rel, a2T_ref[...], preferred_element_type=jnp.float32)
        + c2_ref[...], 0.0)
    s3 = s.reshape(TN, K, C)
    mx = jnp.max(s3, axis=1, keepdims=True)
    e = jnp.exp(s3 - mx)
    w = e / jnp.sum(e, axis=1, keepdims=True)
    feats = (jnp.dot(xj, alphaT_ref[...], preferred_element_type=jnp.float32)
             + balpha_ref[...] + delta).reshape(TN, K, C)
    fm = nm_ref[0] + (1.0 - qm_ref[0])
    feats = feats * fm[:, :, None]
    o_ref[0] = jnp.sum(w * feats, axis=1)


def _run_p3c(ge, gi, query_xyz, qmaskR, nmask, A1T, C1, phiT, bphi, psiT,
             bpsi, A2T, C2, alphaT, balpha):
    grid = (B, N // TN)
    vec = lambda b, i: (0, 0)
    return pl.pallas_call(
        _p3c_body,
        grid=grid,
        in_specs=[
            pl.BlockSpec((1, TN, K, D), lambda b, i: (b, i, 0, 0)),
            pl.BlockSpec((1, TN, D), lambda b, i: (b, i, 0)),
            pl.BlockSpec((1, TN, 3), lambda b, i: (b, i, 0)),
            pl.BlockSpec((1, TN, 1), lambda b, i: (b, i, 0)),
            pl.BlockSpec((1, TN, K), lambda b, i: (b, i, 0)),
            pl.BlockSpec((3, C), vec),
            pl.BlockSpec((1, C), vec),
            pl.BlockSpec((C, C), vec),
            pl.BlockSpec((1, C), vec),
            pl.BlockSpec((C, C), vec),
            pl.BlockSpec((1, C), vec),
            pl.BlockSpec((C, C), vec),
            pl.BlockSpec((1, C), vec),
            pl.BlockSpec((C, C), vec),
            pl.BlockSpec((1, C), vec),
        ],
        out_specs=pl.BlockSpec((1, TN, C), lambda b, i: (b, i, 0)),
        out_shape=jax.ShapeDtypeStruct((B, N, C), jnp.float32),
    )(ge.reshape(B, N, K, D), gi.reshape(B, N, D), query_xyz, qmaskR, nmask,
      A1T, C1, phiT, bphi, psiT, bpsi, A2T, C2, alphaT, balpha)


def _bn_affine(sums, count, g, b):
    mu = sums[0] / count
    var = sums[1] / count - mu * mu
    a = g / jnp.sqrt(var + 1e-5)
    return a, mu


@jax.jit
def kernel(query_xyz, support_xyz, query_mask, support_mask, support_features,
           W_theta1, b_theta1, W_theta2, b_theta2, g_bn_theta, b_bn_theta,
           W_phi, b_phi, W_psi, b_psi, W_alpha, b_alpha,
           W_g1, b_g1, W_g2, b_g2, g_bn_g, b_bn_g):
    f32 = jnp.float32
    sxyzT = jnp.transpose(support_xyz, (0, 2, 1))
    maskR = support_mask.reshape(B, 1, M)
    Weff = jnp.dot(W_theta2, W_theta1)
    beff = jnp.dot(W_theta2, b_theta1) + b_theta2
    Geff = jnp.dot(W_g2, W_g1)
    geffb = jnp.dot(W_g2, b_g1) + b_g2

    idxg, nmask, nng = _run_p1(query_xyz, sxyzT, maskR)

    table = jnp.concatenate(
        [jnp.transpose(support_features, (0, 2, 1)),
         support_xyz,
         jnp.zeros((B, M, D - C - 3), f32)], axis=-1).reshape(B * M, D)
    ge = _make_sc_gather(B * N * K, 512)(table, idxg.reshape(-1))
    gi = _make_sc_gather(B * N, 256)(table, nng.reshape(-1))

    cnt_e = float(B * N * K)
    p1 = _run_p3a(ge, query_xyz, Weff.T, beff.reshape(1, C))
    s = p1.sum(axis=(0, 1))
    a1, mu1 = _bn_affine(s, cnt_e, g_bn_theta, None)
    A1 = a1[:, None] * Weff
    C1 = (a1 * (beff - mu1) + b_bn_theta).reshape(1, C)

    p2 = _run_p3b(ge, gi, query_xyz, A1.T, C1, W_phi.T, b_phi.reshape(1, C),
                  W_psi.T, b_psi.reshape(1, C), Geff.T, geffb.reshape(1, C))
    s2 = p2.sum(axis=(0, 1))
    a2, mu2 = _bn_affine(s2, cnt_e, g_bn_g, None)
    A2 = a2[:, None] * Geff
    C2 = (a2 * (geffb - mu2) + b_bn_g).reshape(1, C)

    out = _run_p3c(ge, gi, query_xyz, query_mask.reshape(B, N, 1), nmask,
                   A1.T, C1, W_phi.T, b_phi.reshape(1, C),
                   W_psi.T, b_psi.reshape(1, C), A2.T, C2,
                   W_alpha.T, b_alpha.reshape(1, C))
    return jnp.transpose(out, (0, 2, 1))
